# baseline (speedup 1.0000x reference)
import functools

import jax
import jax.numpy as jnp
from jax import lax
from jax.experimental import pallas as pl
from jax.experimental.pallas import tpu as pltpu
from jax.experimental.pallas import tpu_sc as plsc

_B, _D = 16384, 1024
_MARGIN = (0.2 + 0.5) / 2.0
_EPS = 1e-8
_K = _B // 2

_NW = 32
_RPW = _B // _NW
_C = 8
_NCH = _RPW // _C
_NST = 5
_SW = _NST * 16


def _sc_stats_body(a_hbm, p_hbm, n_hbm, out_hbm,
                   a0, p0, n0, a1, p1, n1, o_v,
                   sa0, sp0, sn0, sa1, sp1, sn1, so):
    wid = lax.axis_index("s") * 2 + lax.axis_index("c")
    base = wid * _RPW

    def start(c, av, pv, nv, sa, sp, sn):
        rows = base + c * _C
        pltpu.async_copy(a_hbm.at[pl.ds(rows, _C), :], av, sa)
        pltpu.async_copy(p_hbm.at[pl.ds(rows, _C), :], pv, sp)
        pltpu.async_copy(n_hbm.at[pl.ds(rows, _C), :], nv, sn)

    def wait(c, av, pv, nv, sa, sp, sn):
        rows = base + c * _C
        pltpu.make_async_copy(a_hbm.at[pl.ds(rows, _C), :], av, sa).wait()
        pltpu.make_async_copy(p_hbm.at[pl.ds(rows, _C), :], pv, sp).wait()
        pltpu.make_async_copy(n_hbm.at[pl.ds(rows, _C), :], nv, sn).wait()

    def compute(c, av, pv, nv):
        for r in range(_C):
            def sbody(s, accs):
                aa, pp, nn, ap, an = accs
                o = s * 16
                x = av[r, pl.ds(o, 16)]
                y = pv[r, pl.ds(o, 16)]
                z = nv[r, pl.ds(o, 16)]
                return (aa + x * x, pp + y * y, nn + z * z,
                        ap + x * y, an + x * z)

            z16 = jnp.zeros((16,), jnp.float32)
            aa, pp, nn, ap, an = lax.fori_loop(
                0, _D // 16, sbody, (z16, z16, z16, z16, z16), unroll=4)
            row = c * _C + r
            o_v[row, pl.ds(0, 16)] = aa
            o_v[row, pl.ds(16, 16)] = pp
            o_v[row, pl.ds(32, 16)] = nn
            o_v[row, pl.ds(48, 16)] = ap
            o_v[row, pl.ds(64, 16)] = an

    start(0, a0, p0, n0, sa0, sp0, sn0)

    def chunk_pair(g, carry):
        c0 = 2 * g
        start(c0 + 1, a1, p1, n1, sa1, sp1, sn1)
        wait(c0, a0, p0, n0, sa0, sp0, sn0)
        compute(c0, a0, p0, n0)

        c1 = c0 + 1

        @pl.when(c1 + 1 < _NCH)
        def _():
            start(c1 + 1, a0, p0, n0, sa0, sp0, sn0)

        wait(c1, a1, p1, n1, sa1, sp1, sn1)
        compute(c1, a1, p1, n1)
        return carry

    lax.fori_loop(0, _NCH // 2, chunk_pair, 0)

    pltpu.async_copy(o_v, out_hbm.at[pl.ds(base, _RPW), :], so).wait()


_sc_stats = functools.partial(
    pl.kernel,
    mesh=plsc.VectorSubcoreMesh(core_axis_name="c", subcore_axis_name="s"),
    out_type=jax.ShapeDtypeStruct((_B, _SW), jnp.float32),
    scratch_types=[
        pltpu.VMEM((_C, _D), jnp.float32),
        pltpu.VMEM((_C, _D), jnp.float32),
        pltpu.VMEM((_C, _D), jnp.float32),
        pltpu.VMEM((_C, _D), jnp.float32),
        pltpu.VMEM((_C, _D), jnp.float32),
        pltpu.VMEM((_C, _D), jnp.float32),
        pltpu.VMEM((_RPW, _SW), jnp.float32),
        pltpu.SemaphoreType.DMA,
        pltpu.SemaphoreType.DMA,
        pltpu.SemaphoreType.DMA,
        pltpu.SemaphoreType.DMA,
        pltpu.SemaphoreType.DMA,
        pltpu.SemaphoreType.DMA,
        pltpu.SemaphoreType.DMA,
    ],
)(_sc_stats_body)


def _select_kernel(st_ref, out_ref):
    x = st_ref[...]
    aa = jnp.sum(x[:, :, 0:16], axis=-1)
    pp = jnp.sum(x[:, :, 16:32], axis=-1)
    nn = jnp.sum(x[:, :, 32:48], axis=-1)
    ap = jnp.sum(x[:, :, 48:64], axis=-1)
    an = jnp.sum(x[:, :, 64:80], axis=-1)
    na = jnp.maximum(jnp.sqrt(aa), _EPS)
    dpv = 1.0 - ap / (na * jnp.maximum(jnp.sqrt(pp), _EPS))
    dnv = 1.0 - an / (na * jnp.maximum(jnp.sqrt(nn), _EPS))

    u = jax.lax.bitcast_convert_type(dnv, jnp.uint32)
    key = jnp.where((u >> 31) != 0, ~u, u | jnp.uint32(0x80000000))

    def vbody(it, pfx):
        b = (31 - it).astype(jnp.uint32)
        cand = pfx | (jnp.uint32(1) << b)
        cnt = jnp.sum(jnp.where(key >= cand, 1, 0))
        return jnp.where(cnt >= _K, cand, pfx)

    t = jax.lax.fori_loop(0, 32, vbody, jnp.uint32(0))

    gt = key > t
    eq = key == t
    need = _K - jnp.sum(jnp.where(gt, 1, 0))
    nrow, ncol = dnv.shape
    idx = (jax.lax.broadcasted_iota(jnp.int32, (nrow, ncol), 0) * ncol
           + jax.lax.broadcasted_iota(jnp.int32, (nrow, ncol), 1))

    def ibody(_, lohi):
        lo, hi = lohi
        mid = (lo + hi) // 2
        g = jnp.sum(jnp.where(eq & (idx < mid), 1, 0))
        return (jnp.where(g >= need, lo, mid), jnp.where(g >= need, mid, hi))

    _, m = jax.lax.fori_loop(0, 15, ibody, (jnp.int32(0), jnp.int32(_B)))

    sel = gt | (eq & (idx < m))
    loss = jnp.maximum(dpv - dnv + _MARGIN, 0.0)
    total = jnp.sum(jnp.where(sel, loss, 0.0)) / _K
    out_ref[...] = total.reshape(1, 1)


def kernel(anchor, positive, negative):
    stats = _sc_stats(anchor, positive, negative)
    stats3 = stats.reshape(16, _B // 16, _SW)
    out = pl.pallas_call(
        _select_kernel,
        out_specs=pl.BlockSpec((1, 1), lambda: (0, 0)),
        out_shape=jax.ShapeDtypeStruct((1, 1), jnp.float32),
    )(stats3)
    return out[0, 0]


# speedup vs baseline: 1.1783x; 1.1783x over previous
import functools

import jax
import jax.numpy as jnp
from jax import lax
from jax.experimental import pallas as pl
from jax.experimental.pallas import tpu as pltpu
from jax.experimental.pallas import tpu_sc as plsc

_B, _D = 16384, 1024
_MARGIN = (0.2 + 0.5) / 2.0
_EPS = 1e-8
_K = _B // 2

_S = 12288
_R = _B - _S

_TBLK = 1024
_NT = _S // _TBLK

_NW = 32
_RPW = _R // _NW
_C = 8
_NCH = _RPW // _C
_NST = 5
_SW = _NST * 16


def _sc_stats_body(a_hbm, p_hbm, n_hbm, out_hbm,
                   a0, p0, n0, a1, p1, n1, o_v,
                   sa0, sp0, sn0, sa1, sp1, sn1, so):
    wid = lax.axis_index("s") * 2 + lax.axis_index("c")
    base = _S + wid * _RPW

    def start(c, av, pv, nv, sa, sp, sn):
        rows = base + c * _C
        pltpu.async_copy(a_hbm.at[pl.ds(rows, _C), :], av, sa)
        pltpu.async_copy(p_hbm.at[pl.ds(rows, _C), :], pv, sp)
        pltpu.async_copy(n_hbm.at[pl.ds(rows, _C), :], nv, sn)

    def wait(c, av, pv, nv, sa, sp, sn):
        rows = base + c * _C
        pltpu.make_async_copy(a_hbm.at[pl.ds(rows, _C), :], av, sa).wait()
        pltpu.make_async_copy(p_hbm.at[pl.ds(rows, _C), :], pv, sp).wait()
        pltpu.make_async_copy(n_hbm.at[pl.ds(rows, _C), :], nv, sn).wait()

    def compute(c, av, pv, nv):
        for r in range(_C):
            def sbody(s, accs):
                aa, pp, nn, ap, an = accs
                o = s * 16
                x = av[r, pl.ds(o, 16)]
                y = pv[r, pl.ds(o, 16)]
                z = nv[r, pl.ds(o, 16)]
                return (aa + x * x, pp + y * y, nn + z * z,
                        ap + x * y, an + x * z)

            z16 = jnp.zeros((16,), jnp.float32)
            aa, pp, nn, ap, an = lax.fori_loop(
                0, _D // 16, sbody, (z16, z16, z16, z16, z16), unroll=8)
            row = c * _C + r
            o_v[row, pl.ds(0, 16)] = aa
            o_v[row, pl.ds(16, 16)] = pp
            o_v[row, pl.ds(32, 16)] = nn
            o_v[row, pl.ds(48, 16)] = ap
            o_v[row, pl.ds(64, 16)] = an

    start(0, a0, p0, n0, sa0, sp0, sn0)

    def chunk_pair(g, carry):
        c0 = 2 * g
        start(c0 + 1, a1, p1, n1, sa1, sp1, sn1)
        wait(c0, a0, p0, n0, sa0, sp0, sn0)
        compute(c0, a0, p0, n0)

        c1 = c0 + 1

        @pl.when(c1 + 1 < _NCH)
        def _():
            start(c1 + 1, a0, p0, n0, sa0, sp0, sn0)

        wait(c1, a1, p1, n1, sa1, sp1, sn1)
        compute(c1, a1, p1, n1)
        return carry

    lax.fori_loop(0, _NCH // 2, chunk_pair, 0)

    pltpu.async_copy(o_v, out_hbm.at[pl.ds(wid * _RPW, _RPW), :], so).wait()


_sc_stats = functools.partial(
    pl.kernel,
    mesh=plsc.VectorSubcoreMesh(core_axis_name="c", subcore_axis_name="s"),
    out_type=jax.ShapeDtypeStruct((_R, _SW), jnp.float32),
    scratch_types=[
        pltpu.VMEM((_C, _D), jnp.float32),
        pltpu.VMEM((_C, _D), jnp.float32),
        pltpu.VMEM((_C, _D), jnp.float32),
        pltpu.VMEM((_C, _D), jnp.float32),
        pltpu.VMEM((_C, _D), jnp.float32),
        pltpu.VMEM((_C, _D), jnp.float32),
        pltpu.VMEM((_RPW, _SW), jnp.float32),
        pltpu.SemaphoreType.DMA,
        pltpu.SemaphoreType.DMA,
        pltpu.SemaphoreType.DMA,
        pltpu.SemaphoreType.DMA,
        pltpu.SemaphoreType.DMA,
        pltpu.SemaphoreType.DMA,
        pltpu.SemaphoreType.DMA,
    ],
)(_sc_stats_body)


def _tc_dist_kernel(a_ref, p_ref, n_ref, dp_ref, dn_ref):
    a = a_ref[...]
    p = p_ref[...]
    n = n_ref[...]
    aa = jnp.sum(a * a, axis=1)
    pp = jnp.sum(p * p, axis=1)
    nn = jnp.sum(n * n, axis=1)
    ap = jnp.sum(a * p, axis=1)
    an = jnp.sum(a * n, axis=1)
    na = jnp.maximum(jnp.sqrt(aa), _EPS)
    dp = 1.0 - ap / (na * jnp.maximum(jnp.sqrt(pp), _EPS))
    dn = 1.0 - an / (na * jnp.maximum(jnp.sqrt(nn), _EPS))
    dp_ref[...] = dp.reshape(1, 1, _TBLK)
    dn_ref[...] = dn.reshape(1, 1, _TBLK)


def _select_kernel(dp_ref, dn_ref, st_ref, out_ref):
    dp_t = dp_ref[...]
    dn_t = dn_ref[...]

    x = st_ref[...]
    aa = jnp.sum(x[:, :, 0:16], axis=-1)
    pp = jnp.sum(x[:, :, 16:32], axis=-1)
    nn = jnp.sum(x[:, :, 32:48], axis=-1)
    ap = jnp.sum(x[:, :, 48:64], axis=-1)
    an = jnp.sum(x[:, :, 64:80], axis=-1)
    na = jnp.maximum(jnp.sqrt(aa), _EPS)
    dp_s = 1.0 - ap / (na * jnp.maximum(jnp.sqrt(pp), _EPS))
    dn_s = 1.0 - an / (na * jnp.maximum(jnp.sqrt(nn), _EPS))

    def to_key(dn):
        u = jax.lax.bitcast_convert_type(dn, jnp.uint32)
        return jnp.where((u >> 31) != 0, ~u, u | jnp.uint32(0x80000000))

    key_t = to_key(dn_t)
    key_s = to_key(dn_s)

    def vbody(it, pfx):
        b = (31 - it).astype(jnp.uint32)
        cand = pfx | (jnp.uint32(1) << b)
        cnt = (jnp.sum(jnp.where(key_t >= cand, 1, 0))
               + jnp.sum(jnp.where(key_s >= cand, 1, 0)))
        return jnp.where(cnt >= _K, cand, pfx)

    t = jax.lax.fori_loop(0, 32, vbody, jnp.uint32(0))

    gt_t = key_t > t
    eq_t = key_t == t
    gt_s = key_s > t
    eq_s = key_s == t
    need = _K - (jnp.sum(jnp.where(gt_t, 1, 0))
                 + jnp.sum(jnp.where(gt_s, 1, 0)))

    rt, ct = dn_t.shape
    idx_t = (jax.lax.broadcasted_iota(jnp.int32, (rt, ct), 0) * ct
             + jax.lax.broadcasted_iota(jnp.int32, (rt, ct), 1))
    rs, cs = dn_s.shape
    idx_s = (_S + jax.lax.broadcasted_iota(jnp.int32, (rs, cs), 0) * cs
             + jax.lax.broadcasted_iota(jnp.int32, (rs, cs), 1))

    def ibody(_, lohi):
        lo, hi = lohi
        mid = (lo + hi) // 2
        g = (jnp.sum(jnp.where(eq_t & (idx_t < mid), 1, 0))
             + jnp.sum(jnp.where(eq_s & (idx_s < mid), 1, 0)))
        return (jnp.where(g >= need, lo, mid), jnp.where(g >= need, mid, hi))

    _, m = jax.lax.fori_loop(0, 15, ibody, (jnp.int32(0), jnp.int32(_B)))

    sel_t = gt_t | (eq_t & (idx_t < m))
    sel_s = gt_s | (eq_s & (idx_s < m))
    loss_t = jnp.maximum(dp_t - dn_t + _MARGIN, 0.0)
    loss_s = jnp.maximum(dp_s - dn_s + _MARGIN, 0.0)
    total = (jnp.sum(jnp.where(sel_t, loss_t, 0.0))
             + jnp.sum(jnp.where(sel_s, loss_s, 0.0))) / _K
    out_ref[...] = total.reshape(1, 1)


def kernel(anchor, positive, negative):
    stats = _sc_stats(anchor, positive, negative)

    dp_t, dn_t = pl.pallas_call(
        _tc_dist_kernel,
        grid=(_NT,),
        in_specs=[
            pl.BlockSpec((_TBLK, _D), lambda i: (i, 0)),
            pl.BlockSpec((_TBLK, _D), lambda i: (i, 0)),
            pl.BlockSpec((_TBLK, _D), lambda i: (i, 0)),
        ],
        out_specs=[
            pl.BlockSpec((1, 1, _TBLK), lambda i: (i, 0, 0)),
            pl.BlockSpec((1, 1, _TBLK), lambda i: (i, 0, 0)),
        ],
        out_shape=[
            jax.ShapeDtypeStruct((_NT, 1, _TBLK), jnp.float32),
            jax.ShapeDtypeStruct((_NT, 1, _TBLK), jnp.float32),
        ],
    )(anchor[:_S], positive[:_S], negative[:_S])
    dp_t = dp_t.reshape(_NT, _TBLK)
    dn_t = dn_t.reshape(_NT, _TBLK)

    stats3 = stats.reshape(_R // 1024, 1024, _SW)
    out = pl.pallas_call(
        _select_kernel,
        out_specs=pl.BlockSpec((1, 1), lambda: (0, 0)),
        out_shape=jax.ShapeDtypeStruct((1, 1), jnp.float32),
    )(dp_t, dn_t, stats3)
    return out[0, 0]


# speedup vs baseline: 2.0669x; 1.7542x over previous
import functools

import jax
import jax.numpy as jnp
from jax import lax
from jax.experimental import pallas as pl
from jax.experimental.pallas import tpu as pltpu
from jax.experimental.pallas import tpu_sc as plsc

_B, _D = 16384, 1024
_MARGIN = (0.2 + 0.5) / 2.0
_EPS = 1e-8
_K = _B // 2

_S = 12288
_R = _B - _S

_TBLK = 1024
_NT = _S // _TBLK

_NW = 32
_RPW = _R // _NW
_C = 8
_NCH = _RPW // _C
_NST = 5
_SW = _NST * 16


def _sc_stats_body(a_hbm, p_hbm, n_hbm, out_hbm,
                   a0, p0, n0, a1, p1, n1, o_v,
                   sa0, sp0, sn0, sa1, sp1, sn1, so):
    wid = lax.axis_index("s") * 2 + lax.axis_index("c")
    base = _S + wid * _RPW

    def start(c, av, pv, nv, sa, sp, sn):
        rows = base + c * _C
        pltpu.async_copy(a_hbm.at[pl.ds(rows, _C), :], av, sa)
        pltpu.async_copy(p_hbm.at[pl.ds(rows, _C), :], pv, sp)
        pltpu.async_copy(n_hbm.at[pl.ds(rows, _C), :], nv, sn)

    def wait(c, av, pv, nv, sa, sp, sn):
        rows = base + c * _C
        pltpu.make_async_copy(a_hbm.at[pl.ds(rows, _C), :], av, sa).wait()
        pltpu.make_async_copy(p_hbm.at[pl.ds(rows, _C), :], pv, sp).wait()
        pltpu.make_async_copy(n_hbm.at[pl.ds(rows, _C), :], nv, sn).wait()

    def compute(c, av, pv, nv):
        for r in range(_C):
            def sbody(s, accs):
                aa, pp, nn, ap, an = accs
                o = s * 16
                x = av[r, pl.ds(o, 16)]
                y = pv[r, pl.ds(o, 16)]
                z = nv[r, pl.ds(o, 16)]
                return (aa + x * x, pp + y * y, nn + z * z,
                        ap + x * y, an + x * z)

            z16 = jnp.zeros((16,), jnp.float32)
            aa, pp, nn, ap, an = lax.fori_loop(
                0, _D // 16, sbody, (z16, z16, z16, z16, z16), unroll=8)
            row = c * _C + r
            o_v[row, pl.ds(0, 16)] = aa
            o_v[row, pl.ds(16, 16)] = pp
            o_v[row, pl.ds(32, 16)] = nn
            o_v[row, pl.ds(48, 16)] = ap
            o_v[row, pl.ds(64, 16)] = an

    start(0, a0, p0, n0, sa0, sp0, sn0)

    def chunk_pair(g, carry):
        c0 = 2 * g
        start(c0 + 1, a1, p1, n1, sa1, sp1, sn1)
        wait(c0, a0, p0, n0, sa0, sp0, sn0)
        compute(c0, a0, p0, n0)

        c1 = c0 + 1

        @pl.when(c1 + 1 < _NCH)
        def _():
            start(c1 + 1, a0, p0, n0, sa0, sp0, sn0)

        wait(c1, a1, p1, n1, sa1, sp1, sn1)
        compute(c1, a1, p1, n1)
        return carry

    lax.fori_loop(0, _NCH // 2, chunk_pair, 0)

    pltpu.async_copy(o_v, out_hbm.at[pl.ds(wid * _RPW, _RPW), :], so).wait()


_sc_stats = functools.partial(
    pl.kernel,
    mesh=plsc.VectorSubcoreMesh(core_axis_name="c", subcore_axis_name="s"),
    out_type=jax.ShapeDtypeStruct((_R, _SW), jnp.float32),
    scratch_types=[
        pltpu.VMEM((_C, _D), jnp.float32),
        pltpu.VMEM((_C, _D), jnp.float32),
        pltpu.VMEM((_C, _D), jnp.float32),
        pltpu.VMEM((_C, _D), jnp.float32),
        pltpu.VMEM((_C, _D), jnp.float32),
        pltpu.VMEM((_C, _D), jnp.float32),
        pltpu.VMEM((_RPW, _SW), jnp.float32),
        pltpu.SemaphoreType.DMA,
        pltpu.SemaphoreType.DMA,
        pltpu.SemaphoreType.DMA,
        pltpu.SemaphoreType.DMA,
        pltpu.SemaphoreType.DMA,
        pltpu.SemaphoreType.DMA,
        pltpu.SemaphoreType.DMA,
    ],
)(_sc_stats_body)


def _tc_dist_kernel(a_ref, p_ref, n_ref, dp_ref, dn_ref):
    a = a_ref[...]
    p = p_ref[...]
    n = n_ref[...]
    aa = jnp.sum(a * a, axis=1)
    pp = jnp.sum(p * p, axis=1)
    nn = jnp.sum(n * n, axis=1)
    ap = jnp.sum(a * p, axis=1)
    an = jnp.sum(a * n, axis=1)
    na = jnp.maximum(jnp.sqrt(aa), _EPS)
    dp = 1.0 - ap / (na * jnp.maximum(jnp.sqrt(pp), _EPS))
    dn = 1.0 - an / (na * jnp.maximum(jnp.sqrt(nn), _EPS))
    dp_ref[...] = dp.reshape(1, 1, _TBLK)
    dn_ref[...] = dn.reshape(1, 1, _TBLK)


def _select_kernel(dp_ref, dn_ref, st_ref, out_ref):
    dp_t = dp_ref[...]
    dn_t = dn_ref[...]

    x = st_ref[...]
    aa = jnp.sum(x[:, :, 0:16], axis=-1)
    pp = jnp.sum(x[:, :, 16:32], axis=-1)
    nn = jnp.sum(x[:, :, 32:48], axis=-1)
    ap = jnp.sum(x[:, :, 48:64], axis=-1)
    an = jnp.sum(x[:, :, 64:80], axis=-1)
    na = jnp.maximum(jnp.sqrt(aa), _EPS)
    dp_s = 1.0 - ap / (na * jnp.maximum(jnp.sqrt(pp), _EPS))
    dn_s = 1.0 - an / (na * jnp.maximum(jnp.sqrt(nn), _EPS))

    def to_key(dn):
        u = jax.lax.bitcast_convert_type(dn, jnp.uint32)
        return jnp.where((u >> 31) != 0, ~u, u | jnp.uint32(0x80000000))

    key_t = to_key(dn_t)
    key_s = to_key(dn_s)

    def vbody(it, pfx):
        b = (31 - it).astype(jnp.uint32)
        cand = pfx | (jnp.uint32(1) << b)
        cnt = (jnp.sum(jnp.where(key_t >= cand, 1, 0))
               + jnp.sum(jnp.where(key_s >= cand, 1, 0)))
        return jnp.where(cnt >= _K, cand, pfx)

    t = jax.lax.fori_loop(0, 32, vbody, jnp.uint32(0))

    gt_t = key_t > t
    eq_t = key_t == t
    gt_s = key_s > t
    eq_s = key_s == t
    need = _K - (jnp.sum(jnp.where(gt_t, 1, 0))
                 + jnp.sum(jnp.where(gt_s, 1, 0)))

    rt, ct = dn_t.shape
    idx_t = (jax.lax.broadcasted_iota(jnp.int32, (rt, ct), 0) * ct
             + jax.lax.broadcasted_iota(jnp.int32, (rt, ct), 1))
    rs, cs = dn_s.shape
    idx_s = (_S + jax.lax.broadcasted_iota(jnp.int32, (rs, cs), 0) * cs
             + jax.lax.broadcasted_iota(jnp.int32, (rs, cs), 1))

    def ibody(_, lohi):
        lo, hi = lohi
        mid = (lo + hi) // 2
        g = (jnp.sum(jnp.where(eq_t & (idx_t < mid), 1, 0))
             + jnp.sum(jnp.where(eq_s & (idx_s < mid), 1, 0)))
        return (jnp.where(g >= need, lo, mid), jnp.where(g >= need, mid, hi))

    _, m = jax.lax.fori_loop(0, 15, ibody, (jnp.int32(0), jnp.int32(_B)))

    sel_t = gt_t | (eq_t & (idx_t < m))
    sel_s = gt_s | (eq_s & (idx_s < m))
    loss_t = jnp.maximum(dp_t - dn_t + _MARGIN, 0.0)
    loss_s = jnp.maximum(dp_s - dn_s + _MARGIN, 0.0)
    total = (jnp.sum(jnp.where(sel_t, loss_t, 0.0))
             + jnp.sum(jnp.where(sel_s, loss_s, 0.0))) / _K
    out_ref[...] = total.reshape(1, 1)


def kernel(anchor, positive, negative):
    stats = _sc_stats(anchor, positive, negative)

    dp_t, dn_t = pl.pallas_call(
        _tc_dist_kernel,
        grid=(_NT,),
        in_specs=[
            pl.BlockSpec((_TBLK, _D), lambda i: (i, 0)),
            pl.BlockSpec((_TBLK, _D), lambda i: (i, 0)),
            pl.BlockSpec((_TBLK, _D), lambda i: (i, 0)),
        ],
        out_specs=[
            pl.BlockSpec((1, 1, _TBLK), lambda i: (i, 0, 0)),
            pl.BlockSpec((1, 1, _TBLK), lambda i: (i, 0, 0)),
        ],
        out_shape=[
            jax.ShapeDtypeStruct((_NT, 1, _TBLK), jnp.float32),
            jax.ShapeDtypeStruct((_NT, 1, _TBLK), jnp.float32),
        ],
    )(anchor, positive, negative)
    dp_t = dp_t.reshape(_NT, _TBLK)
    dn_t = dn_t.reshape(_NT, _TBLK)

    stats3 = stats.reshape(_R // 1024, 1024, _SW)
    out = pl.pallas_call(
        _select_kernel,
        out_specs=pl.BlockSpec((1, 1), lambda: (0, 0)),
        out_shape=jax.ShapeDtypeStruct((1, 1), jnp.float32),
    )(dp_t, dn_t, stats3)
    return out[0, 0]


# speedup vs baseline: 2.0684x; 1.0008x over previous
import functools

import jax
import jax.numpy as jnp
from jax import lax
from jax.experimental import pallas as pl
from jax.experimental.pallas import tpu as pltpu
from jax.experimental.pallas import tpu_sc as plsc

_B, _D = 16384, 1024
_MARGIN = (0.2 + 0.5) / 2.0
_EPS = 1e-8
_K = _B // 2

_S = 12288
_R = _B - _S

_TBLK = 1024
_NT = _S // _TBLK

_NW = 32
_RPW = _R // _NW
_C = 8
_NCH = _RPW // _C
_NST = 5
_SW = _NST * 16


def _sc_stats_body(a_hbm, p_hbm, n_hbm, out_hbm,
                   a0, p0, n0, a1, p1, n1, o_v,
                   sa0, sp0, sn0, sa1, sp1, sn1, so):
    wid = lax.axis_index("s") * 2 + lax.axis_index("c")
    base = _S + wid * _RPW

    def start(c, av, pv, nv, sa, sp, sn):
        rows = base + c * _C
        pltpu.async_copy(a_hbm.at[pl.ds(rows, _C), :], av, sa)
        pltpu.async_copy(p_hbm.at[pl.ds(rows, _C), :], pv, sp)
        pltpu.async_copy(n_hbm.at[pl.ds(rows, _C), :], nv, sn)

    def wait(c, av, pv, nv, sa, sp, sn):
        rows = base + c * _C
        pltpu.make_async_copy(a_hbm.at[pl.ds(rows, _C), :], av, sa).wait()
        pltpu.make_async_copy(p_hbm.at[pl.ds(rows, _C), :], pv, sp).wait()
        pltpu.make_async_copy(n_hbm.at[pl.ds(rows, _C), :], nv, sn).wait()

    def compute(c, av, pv, nv):
        for r in range(_C):
            def sbody(s, accs):
                aa, pp, nn, ap, an = accs
                o = s * 16
                x = av[r, pl.ds(o, 16)]
                y = pv[r, pl.ds(o, 16)]
                z = nv[r, pl.ds(o, 16)]
                return (aa + x * x, pp + y * y, nn + z * z,
                        ap + x * y, an + x * z)

            z16 = jnp.zeros((16,), jnp.float32)
            aa, pp, nn, ap, an = lax.fori_loop(
                0, _D // 16, sbody, (z16, z16, z16, z16, z16), unroll=8)
            row = c * _C + r
            o_v[row, pl.ds(0, 16)] = aa
            o_v[row, pl.ds(16, 16)] = pp
            o_v[row, pl.ds(32, 16)] = nn
            o_v[row, pl.ds(48, 16)] = ap
            o_v[row, pl.ds(64, 16)] = an

    start(0, a0, p0, n0, sa0, sp0, sn0)

    def chunk_pair(g, carry):
        c0 = 2 * g
        start(c0 + 1, a1, p1, n1, sa1, sp1, sn1)
        wait(c0, a0, p0, n0, sa0, sp0, sn0)
        compute(c0, a0, p0, n0)

        c1 = c0 + 1

        @pl.when(c1 + 1 < _NCH)
        def _():
            start(c1 + 1, a0, p0, n0, sa0, sp0, sn0)

        wait(c1, a1, p1, n1, sa1, sp1, sn1)
        compute(c1, a1, p1, n1)
        return carry

    lax.fori_loop(0, _NCH // 2, chunk_pair, 0)

    pltpu.async_copy(o_v, out_hbm.at[pl.ds(wid * _RPW, _RPW), :], so).wait()


_sc_stats = functools.partial(
    pl.kernel,
    mesh=plsc.VectorSubcoreMesh(core_axis_name="c", subcore_axis_name="s"),
    out_type=jax.ShapeDtypeStruct((_R, _SW), jnp.float32),
    scratch_types=[
        pltpu.VMEM((_C, _D), jnp.float32),
        pltpu.VMEM((_C, _D), jnp.float32),
        pltpu.VMEM((_C, _D), jnp.float32),
        pltpu.VMEM((_C, _D), jnp.float32),
        pltpu.VMEM((_C, _D), jnp.float32),
        pltpu.VMEM((_C, _D), jnp.float32),
        pltpu.VMEM((_RPW, _SW), jnp.float32),
        pltpu.SemaphoreType.DMA,
        pltpu.SemaphoreType.DMA,
        pltpu.SemaphoreType.DMA,
        pltpu.SemaphoreType.DMA,
        pltpu.SemaphoreType.DMA,
        pltpu.SemaphoreType.DMA,
        pltpu.SemaphoreType.DMA,
    ],
)(_sc_stats_body)


def _tc_dist_kernel(a_ref, p_ref, n_ref, dp_ref, dn_ref):
    a = a_ref[...]
    p = p_ref[...]
    n = n_ref[...]
    aa = jnp.sum(a * a, axis=1)
    pp = jnp.sum(p * p, axis=1)
    nn = jnp.sum(n * n, axis=1)
    ap = jnp.sum(a * p, axis=1)
    an = jnp.sum(a * n, axis=1)
    na = jnp.maximum(jnp.sqrt(aa), _EPS)
    dp = 1.0 - ap / (na * jnp.maximum(jnp.sqrt(pp), _EPS))
    dn = 1.0 - an / (na * jnp.maximum(jnp.sqrt(nn), _EPS))
    dp_ref[...] = dp.reshape(1, 1, _TBLK)
    dn_ref[...] = dn.reshape(1, 1, _TBLK)


def _select_kernel(dp_ref, dn_ref, st_ref, out_ref):
    dp_t = dp_ref[...]
    dn_t = dn_ref[...]

    x = st_ref[...]
    aa = jnp.sum(x[:, :, 0:16], axis=-1)
    pp = jnp.sum(x[:, :, 16:32], axis=-1)
    nn = jnp.sum(x[:, :, 32:48], axis=-1)
    ap = jnp.sum(x[:, :, 48:64], axis=-1)
    an = jnp.sum(x[:, :, 64:80], axis=-1)
    na = jnp.maximum(jnp.sqrt(aa), _EPS)
    dp_s = 1.0 - ap / (na * jnp.maximum(jnp.sqrt(pp), _EPS))
    dn_s = 1.0 - an / (na * jnp.maximum(jnp.sqrt(nn), _EPS))

    def to_key(dn):
        u = jax.lax.bitcast_convert_type(dn, jnp.uint32)
        return jnp.where((u >> 31) != 0, ~u, u | jnp.uint32(0x80000000))

    key_t = to_key(dn_t)
    key_s = to_key(dn_s)

    def vbody(it, pfx):
        b = (31 - it).astype(jnp.uint32)
        cand = pfx | (jnp.uint32(1) << b)
        cnt = (jnp.sum(jnp.where(key_t >= cand, 1, 0))
               + jnp.sum(jnp.where(key_s >= cand, 1, 0)))
        return jnp.where(cnt >= _K, cand, pfx)

    t = jax.lax.fori_loop(0, 32, vbody, jnp.uint32(0))

    gt_t = key_t > t
    eq_t = key_t == t
    gt_s = key_s > t
    eq_s = key_s == t
    need = _K - (jnp.sum(jnp.where(gt_t, 1, 0))
                 + jnp.sum(jnp.where(gt_s, 1, 0)))

    rt, ct = dn_t.shape
    idx_t = (jax.lax.broadcasted_iota(jnp.int32, (rt, ct), 0) * ct
             + jax.lax.broadcasted_iota(jnp.int32, (rt, ct), 1))
    rs, cs = dn_s.shape
    idx_s = (_S + jax.lax.broadcasted_iota(jnp.int32, (rs, cs), 0) * cs
             + jax.lax.broadcasted_iota(jnp.int32, (rs, cs), 1))

    def ibody(_, lohi):
        lo, hi = lohi
        mid = (lo + hi) // 2
        g = (jnp.sum(jnp.where(eq_t & (idx_t < mid), 1, 0))
             + jnp.sum(jnp.where(eq_s & (idx_s < mid), 1, 0)))
        return (jnp.where(g >= need, lo, mid), jnp.where(g >= need, mid, hi))

    _, m = jax.lax.fori_loop(0, 15, ibody, (jnp.int32(0), jnp.int32(_B)))

    sel_t = gt_t | (eq_t & (idx_t < m))
    sel_s = gt_s | (eq_s & (idx_s < m))
    loss_t = jnp.maximum(dp_t - dn_t + _MARGIN, 0.0)
    loss_s = jnp.maximum(dp_s - dn_s + _MARGIN, 0.0)
    total = (jnp.sum(jnp.where(sel_t, loss_t, 0.0))
             + jnp.sum(jnp.where(sel_s, loss_s, 0.0))) / _K
    out_ref[...] = total.reshape(1, 1)


def kernel(anchor, positive, negative):
    dp_t, dn_t = pl.pallas_call(
        _tc_dist_kernel,
        grid=(_NT,),
        in_specs=[
            pl.BlockSpec((_TBLK, _D), lambda i: (i, 0)),
            pl.BlockSpec((_TBLK, _D), lambda i: (i, 0)),
            pl.BlockSpec((_TBLK, _D), lambda i: (i, 0)),
        ],
        out_specs=[
            pl.BlockSpec((1, 1, _TBLK), lambda i: (i, 0, 0)),
            pl.BlockSpec((1, 1, _TBLK), lambda i: (i, 0, 0)),
        ],
        out_shape=[
            jax.ShapeDtypeStruct((_NT, 1, _TBLK), jnp.float32),
            jax.ShapeDtypeStruct((_NT, 1, _TBLK), jnp.float32),
        ],
    )(anchor, positive, negative)
    dp_t = dp_t.reshape(_NT, _TBLK)
    dn_t = dn_t.reshape(_NT, _TBLK)

    stats = _sc_stats(anchor, positive, negative)

    stats3 = stats.reshape(_R // 1024, 1024, _SW)
    out = pl.pallas_call(
        _select_kernel,
        out_specs=pl.BlockSpec((1, 1), lambda: (0, 0)),
        out_shape=jax.ShapeDtypeStruct((1, 1), jnp.float32),
    )(dp_t, dn_t, stats3)
    return out[0, 0]


# speedup vs baseline: 3.7344x; 1.8055x over previous
import jax
import jax.numpy as jnp
from jax.experimental import pallas as pl
from jax.experimental.pallas import tpu as pltpu

_B, _D = 16384, 1024
_MARGIN = (0.2 + 0.5) / 2.0
_EPS = 1e-8
_K = _B // 2
_BLK = 1024
_NBLK = _B // _BLK


def _tl_kernel(a_ref, p_ref, n_ref, out_ref, dp_ref, dn_ref):
    i = pl.program_id(0)
    a = a_ref[...]
    p = p_ref[...]
    n = n_ref[...]
    aa = jnp.sum(a * a, axis=1)
    pp = jnp.sum(p * p, axis=1)
    nn = jnp.sum(n * n, axis=1)
    ap = jnp.sum(a * p, axis=1)
    an = jnp.sum(a * n, axis=1)
    na = jnp.maximum(jnp.sqrt(aa), _EPS)
    dp = 1.0 - ap / (na * jnp.maximum(jnp.sqrt(pp), _EPS))
    dn = 1.0 - an / (na * jnp.maximum(jnp.sqrt(nn), _EPS))
    dp_ref[pl.ds(i, 1), :] = dp.reshape(1, _BLK)
    dn_ref[pl.ds(i, 1), :] = dn.reshape(1, _BLK)

    @pl.when(i == _NBLK - 1)
    def _select():
        dnv = dn_ref[...]
        dpv = dp_ref[...]
        u = jax.lax.bitcast_convert_type(dnv, jnp.uint32)
        key = jnp.where((u >> 31) != 0, ~u, u | jnp.uint32(0x80000000))

        def vbody(it, pfx):
            b = (31 - it).astype(jnp.uint32)
            cand = pfx | (jnp.uint32(1) << b)
            cnt = jnp.sum(jnp.where(key >= cand, 1, 0))
            return jnp.where(cnt >= _K, cand, pfx)

        t = jax.lax.fori_loop(0, 32, vbody, jnp.uint32(0))

        gt = key > t
        eq = key == t
        need = _K - jnp.sum(jnp.where(gt, 1, 0))
        idx = (jax.lax.broadcasted_iota(jnp.int32, (_NBLK, _BLK), 0) * _BLK
               + jax.lax.broadcasted_iota(jnp.int32, (_NBLK, _BLK), 1))

        def ibody(_, lohi):
            lo, hi = lohi
            mid = (lo + hi) // 2
            g = jnp.sum(jnp.where(eq & (idx < mid), 1, 0))
            return (jnp.where(g >= need, lo, mid), jnp.where(g >= need, mid, hi))

        _, m = jax.lax.fori_loop(0, 15, ibody, (jnp.int32(0), jnp.int32(_B)))

        sel = gt | (eq & (idx < m))
        loss = jnp.maximum(dpv - dnv + _MARGIN, 0.0)
        total = jnp.sum(jnp.where(sel, loss, 0.0)) / _K
        out_ref[...] = total.reshape(1, 1)


def kernel(anchor, positive, negative):
    out = pl.pallas_call(
        _tl_kernel,
        grid=(_NBLK,),
        in_specs=[
            pl.BlockSpec((_BLK, _D), lambda i: (i, 0)),
            pl.BlockSpec((_BLK, _D), lambda i: (i, 0)),
            pl.BlockSpec((_BLK, _D), lambda i: (i, 0)),
        ],
        out_specs=pl.BlockSpec((1, 1), lambda i: (0, 0)),
        out_shape=jax.ShapeDtypeStruct((1, 1), jnp.float32),
        scratch_shapes=[
            pltpu.VMEM((_NBLK, _BLK), jnp.float32),
            pltpu.VMEM((_NBLK, _BLK), jnp.float32),
        ],
        compiler_params=pltpu.CompilerParams(
            dimension_semantics=("arbitrary",),
        ),
    )(anchor, positive, negative)
    return out[0, 0]
